# speedup vs baseline: 3.3539x; 3.3539x over previous
import functools

import jax
import jax.numpy as jnp
from jax import lax
from jax.experimental import pallas as pl
from jax.experimental.pallas import tpu as pltpu
from jax.experimental.pallas import tpu_sc as plsc

N = 10000
D = 128
E = 320000

NC = 2
NS = 16
NW = NC * NS
L = 16

CH = 128
EPW = 10240
NCH = EPW // CH
EPAD = EPW * NW
NACC = 10240
RPT = NACC // NS

BLK = 1000

_mesh = plsc.VectorSubcoreMesh(core_axis_name="c", subcore_axis_name="s")


@functools.partial(
    pl.kernel,
    out_type=(
        jax.ShapeDtypeStruct((NC, NACC, D), jnp.float32),
        jax.ShapeDtypeStruct((NC, NACC), jnp.float32),
    ),
    mesh=_mesh,
    scratch_types=[
        pltpu.VMEM((NCH, CH), jnp.int32),
        pltpu.VMEM((NCH, CH), jnp.int32),
        pltpu.VMEM((CH, D), jnp.float32),
        pltpu.VMEM((L, D), jnp.float32),
        pltpu.VMEM((RPT,), jnp.float32),
        pltpu.VMEM((CH,), jnp.float32),
        pltpu.VMEM_SHARED((NACC, D), jnp.float32),
        pltpu.VMEM_SHARED((NACC,), jnp.float32),
        pltpu.SemaphoreType.DMA,
    ],
)
def _sc_scatter(p_hbm, src_hbm, dst_hbm, acc_out, cnt_out,
                src_v, dst_v, rows_v, zb, czb, ones_v, acc_sh, cnt_sh, sem):
    cid = lax.axis_index("c")
    sid = lax.axis_index("s")
    wid = sid * NC + cid
    base = sid * RPT

    zero16 = jnp.zeros((L,), jnp.float32)
    one16 = jnp.ones((L,), jnp.float32)
    for i in range(L):
        for c in range(D // L):
            zb[i, pl.ds(c * L, L)] = zero16
    for c in range(CH // L):
        ones_v[pl.ds(c * L, L)] = one16

    def czfill(k, carry):
        czb[pl.ds(k * L, L)] = zero16
        return carry
    lax.fori_loop(0, RPT // L, czfill, 0)

    def zfill(k, carry):
        pltpu.sync_copy(zb, acc_sh.at[pl.ds(base + k * L, L)])
        return carry
    lax.fori_loop(0, RPT // L, zfill, 0)
    pltpu.sync_copy(czb, cnt_sh.at[pl.ds(base, RPT)])

    pltpu.sync_copy(src_hbm.at[wid], src_v)
    pltpu.sync_copy(dst_hbm.at[wid], dst_v)
    plsc.subcore_barrier()

    def body(j, carry):
        pltpu.async_copy(p_hbm.at[src_v.at[j]], rows_v, sem).wait()
        pltpu.sync_copy(rows_v, acc_sh.at[dst_v.at[j]], add=True)
        pltpu.sync_copy(ones_v, cnt_sh.at[dst_v.at[j]], add=True)
        return carry
    lax.fori_loop(0, NCH, body, 0)
    plsc.subcore_barrier()

    pltpu.sync_copy(acc_sh.at[pl.ds(base, RPT)],
                    acc_out.at[cid, pl.ds(base, RPT)])
    pltpu.sync_copy(cnt_sh.at[pl.ds(base, RPT)],
                    cnt_out.at[cid, pl.ds(base, RPT)])


def _pre_body(x_ref, wl_ref, wr_ref, b_ref, p_ref, r_ref):
    xb = x_ref[...]
    p_ref[...] = jnp.dot(xb, wl_ref[...], preferred_element_type=jnp.float32)
    r_ref[...] = (jnp.dot(xb, wr_ref[...], preferred_element_type=jnp.float32)
                  + b_ref[...])


def _pre(x, wl, wr, b):
    return pl.pallas_call(
        _pre_body,
        grid=(N // BLK,),
        in_specs=[
            pl.BlockSpec((BLK, D), lambda i: (i, 0)),
            pl.BlockSpec((D, D), lambda i: (0, 0)),
            pl.BlockSpec((D, D), lambda i: (0, 0)),
            pl.BlockSpec((1, D), lambda i: (0, 0)),
        ],
        out_specs=[
            pl.BlockSpec((BLK, D), lambda i: (i, 0)),
            pl.BlockSpec((BLK, D), lambda i: (i, 0)),
        ],
        out_shape=[jax.ShapeDtypeStruct((N, D), jnp.float32)] * 2,
    )(x, wl, wr, b)


def _mid_body(acc_ref, cnt_ref, r_ref, wl_ref, wr_ref, b_ref, p_ref, q_ref):
    s = acc_ref[0] + acc_ref[1]
    c = cnt_ref[0] + cnt_ref[1]
    mean = s / jnp.maximum(c, 1.0)
    h = jnp.maximum(mean + r_ref[...], 0.0)
    p_ref[...] = jnp.dot(h, wl_ref[...], preferred_element_type=jnp.float32)
    q_ref[...] = (jnp.dot(h, wr_ref[...], preferred_element_type=jnp.float32)
                  + b_ref[...])


def _mid(acc, cnt3, r1, wl, wr, b):
    return pl.pallas_call(
        _mid_body,
        grid=(N // BLK,),
        in_specs=[
            pl.BlockSpec((2, BLK, D), lambda i: (0, i, 0)),
            pl.BlockSpec((2, BLK, 1), lambda i: (0, i, 0)),
            pl.BlockSpec((BLK, D), lambda i: (i, 0)),
            pl.BlockSpec((D, D), lambda i: (0, 0)),
            pl.BlockSpec((D, D), lambda i: (0, 0)),
            pl.BlockSpec((1, D), lambda i: (0, 0)),
        ],
        out_specs=[
            pl.BlockSpec((BLK, D), lambda i: (i, 0)),
            pl.BlockSpec((BLK, D), lambda i: (i, 0)),
        ],
        out_shape=[jax.ShapeDtypeStruct((N, D), jnp.float32)] * 2,
    )(acc, cnt3, r1, wl, wr, b)


def _post_body(acc_ref, cnt_ref, q_ref, o_ref):
    s = acc_ref[0] + acc_ref[1]
    c = cnt_ref[0] + cnt_ref[1]
    o_ref[...] = s / jnp.maximum(c, 1.0) + q_ref[...]


def _post(acc, cnt3, q):
    return pl.pallas_call(
        _post_body,
        grid=(N // BLK,),
        in_specs=[
            pl.BlockSpec((2, BLK, D), lambda i: (0, i, 0)),
            pl.BlockSpec((2, BLK, 1), lambda i: (0, i, 0)),
            pl.BlockSpec((BLK, D), lambda i: (i, 0)),
        ],
        out_specs=pl.BlockSpec((BLK, D), lambda i: (i, 0)),
        out_shape=jax.ShapeDtypeStruct((N, D), jnp.float32),
    )(acc, cnt3, q)


def kernel(x, edge_index, W1l, b1, W1r, W2l, b2, W2r):
    ei = edge_index.astype(jnp.int32)
    pad = EPAD - E
    src = jnp.concatenate([ei[0], jnp.zeros((pad,), jnp.int32)])
    dst = jnp.concatenate([ei[1], jnp.full((pad,), N, jnp.int32)])
    srcw = src.reshape(NW, NCH, CH)
    dstw = dst.reshape(NW, NCH, CH)
    b1r = b1.reshape(1, D)
    b2r = b2.reshape(1, D)

    p1, r1 = _pre(x, W1l, W1r, b1r)
    acc1, cnt = _sc_scatter(p1, srcw, dstw)
    cnt3 = cnt.reshape(NC, NACC, 1)
    p2, r2 = _mid(acc1, cnt3, r1, W2l, W2r, b2r)
    acc2, _ = _sc_scatter(p2, srcw, dstw)
    return _post(acc2, cnt3, r2)


# speedup vs baseline: 3.9703x; 1.1838x over previous
import functools

import jax
import jax.numpy as jnp
from jax import lax
from jax.experimental import pallas as pl
from jax.experimental.pallas import tpu as pltpu
from jax.experimental.pallas import tpu_sc as plsc

N = 10000
D = 128
E = 320000

NC = 2
NS = 16
NW = NC * NS
L = 16

CH = 128
EPW = 10240
NCH = EPW // CH
EPAD = EPW * NW
NACC = 10240
RPT = NACC // NS

BLK = 1000

_mesh = plsc.VectorSubcoreMesh(core_axis_name="c", subcore_axis_name="s")


def _build_sc(with_cnt):
    if with_cnt:
        out_type = (
            jax.ShapeDtypeStruct((NC, NACC, D), jnp.float32),
            jax.ShapeDtypeStruct((NC, NACC), jnp.float32),
        )
    else:
        out_type = jax.ShapeDtypeStruct((NC, NACC, D), jnp.float32)
    scratch = [
        pltpu.VMEM((NCH // 2, CH), jnp.int32),
        pltpu.VMEM((NCH // 2, CH), jnp.int32),
        pltpu.VMEM((CH, D), jnp.float32),
        pltpu.VMEM((CH, D), jnp.float32),
        pltpu.VMEM((L, D), jnp.float32),
        pltpu.VMEM_SHARED((NACC, D), jnp.float32),
        pltpu.SemaphoreType.DMA,
        pltpu.SemaphoreType.DMA,
    ]
    if with_cnt:
        scratch += [
            pltpu.VMEM((RPT,), jnp.float32),
            pltpu.VMEM((CH,), jnp.float32),
            pltpu.VMEM_SHARED((NACC,), jnp.float32),
        ]

    def body(p_hbm, src_hbm, dst_hbm, *rest):
        if with_cnt:
            (acc_out, cnt_out, src_v, dst_v, rows_a, rows_b, zb, acc_sh,
             sem_a, sem_b, czb, ones_v, cnt_sh) = rest
        else:
            (acc_out, src_v, dst_v, rows_a, rows_b, zb, acc_sh,
             sem_a, sem_b) = rest
        cid = lax.axis_index("c")
        sid = lax.axis_index("s")
        wid = sid * NC + cid
        base = sid * RPT

        zero16 = jnp.zeros((L,), jnp.float32)
        for i in range(L):
            for c in range(D // L):
                zb[i, pl.ds(c * L, L)] = zero16
        if with_cnt:
            one16 = jnp.ones((L,), jnp.float32)
            for c in range(CH // L):
                ones_v[pl.ds(c * L, L)] = one16

            def czfill(k, carry):
                czb[pl.ds(k * L, L)] = zero16
                return carry
            lax.fori_loop(0, RPT // L, czfill, 0)

        def zfill(k, carry):
            pltpu.sync_copy(zb, acc_sh.at[pl.ds(base + k * L, L)])
            return carry
        lax.fori_loop(0, RPT // L, zfill, 0)
        if with_cnt:
            pltpu.sync_copy(czb, cnt_sh.at[pl.ds(base, RPT)])

        plsc.subcore_barrier()

        def gather(j, buf, sem):
            return pltpu.async_copy(p_hbm.at[src_v.at[j]], buf, sem)

        def scatter(j, buf):
            pltpu.sync_copy(buf, acc_sh.at[dst_v.at[j]], add=True)
            if with_cnt:
                pltpu.sync_copy(ones_v, cnt_sh.at[dst_v.at[j]], add=True)

        def wait(j, buf, sem):
            pltpu.make_async_copy(p_hbm.at[src_v.at[j]], buf, sem).wait()

        HP = NCH // 2
        for ph in range(2):
            pltpu.sync_copy(src_hbm.at[wid, pl.ds(ph * HP, HP)], src_v)
            pltpu.sync_copy(dst_hbm.at[wid, pl.ds(ph * HP, HP)], dst_v)
            gather(0, rows_a, sem_a)

            def loop(jj, carry):
                j = 2 * jj
                gather(j + 1, rows_b, sem_b)
                wait(j, rows_a, sem_a)
                scatter(j, rows_a)
                gather(j + 2, rows_a, sem_a)
                wait(j + 1, rows_b, sem_b)
                scatter(j + 1, rows_b)
                return carry
            lax.fori_loop(0, HP // 2 - 1, loop, 0)

            gather(HP - 1, rows_b, sem_b)
            wait(HP - 2, rows_a, sem_a)
            scatter(HP - 2, rows_a)
            wait(HP - 1, rows_b, sem_b)
            scatter(HP - 1, rows_b)
        plsc.subcore_barrier()

        pltpu.sync_copy(acc_sh.at[pl.ds(base, RPT)],
                        acc_out.at[cid, pl.ds(base, RPT)])
        if with_cnt:
            pltpu.sync_copy(cnt_sh.at[pl.ds(base, RPT)],
                            cnt_out.at[cid, pl.ds(base, RPT)])

    return pl.kernel(body, out_type=out_type, mesh=_mesh,
                     scratch_types=scratch)


_sc_scatter_cnt = _build_sc(True)
_sc_scatter_nocnt = _build_sc(False)


def _pre_body(x_ref, wl_ref, wr_ref, b_ref, p_ref, r_ref):
    xb = x_ref[...]
    p_ref[...] = jnp.dot(xb, wl_ref[...], preferred_element_type=jnp.float32)
    r_ref[...] = (jnp.dot(xb, wr_ref[...], preferred_element_type=jnp.float32)
                  + b_ref[...])


def _pre(x, wl, wr, b):
    return pl.pallas_call(
        _pre_body,
        grid=(N // BLK,),
        in_specs=[
            pl.BlockSpec((BLK, D), lambda i: (i, 0)),
            pl.BlockSpec((D, D), lambda i: (0, 0)),
            pl.BlockSpec((D, D), lambda i: (0, 0)),
            pl.BlockSpec((1, D), lambda i: (0, 0)),
        ],
        out_specs=[
            pl.BlockSpec((BLK, D), lambda i: (i, 0)),
            pl.BlockSpec((BLK, D), lambda i: (i, 0)),
        ],
        out_shape=[jax.ShapeDtypeStruct((N, D), jnp.float32)] * 2,
    )(x, wl, wr, b)


def _mid_body(acc_ref, cnt_ref, r_ref, wl_ref, wr_ref, b_ref, p_ref, q_ref):
    s = acc_ref[0] + acc_ref[1]
    c = cnt_ref[0] + cnt_ref[1]
    mean = s / jnp.maximum(c, 1.0)
    h = jnp.maximum(mean + r_ref[...], 0.0)
    p_ref[...] = jnp.dot(h, wl_ref[...], preferred_element_type=jnp.float32)
    q_ref[...] = (jnp.dot(h, wr_ref[...], preferred_element_type=jnp.float32)
                  + b_ref[...])


def _mid(acc, cnt3, r1, wl, wr, b):
    return pl.pallas_call(
        _mid_body,
        grid=(N // BLK,),
        in_specs=[
            pl.BlockSpec((2, BLK, D), lambda i: (0, i, 0)),
            pl.BlockSpec((2, BLK, 1), lambda i: (0, i, 0)),
            pl.BlockSpec((BLK, D), lambda i: (i, 0)),
            pl.BlockSpec((D, D), lambda i: (0, 0)),
            pl.BlockSpec((D, D), lambda i: (0, 0)),
            pl.BlockSpec((1, D), lambda i: (0, 0)),
        ],
        out_specs=[
            pl.BlockSpec((BLK, D), lambda i: (i, 0)),
            pl.BlockSpec((BLK, D), lambda i: (i, 0)),
        ],
        out_shape=[jax.ShapeDtypeStruct((N, D), jnp.float32)] * 2,
    )(acc, cnt3, r1, wl, wr, b)


def _post_body(acc_ref, cnt_ref, q_ref, o_ref):
    s = acc_ref[0] + acc_ref[1]
    c = cnt_ref[0] + cnt_ref[1]
    o_ref[...] = s / jnp.maximum(c, 1.0) + q_ref[...]


def _post(acc, cnt3, q):
    return pl.pallas_call(
        _post_body,
        grid=(N // BLK,),
        in_specs=[
            pl.BlockSpec((2, BLK, D), lambda i: (0, i, 0)),
            pl.BlockSpec((2, BLK, 1), lambda i: (0, i, 0)),
            pl.BlockSpec((BLK, D), lambda i: (i, 0)),
        ],
        out_specs=pl.BlockSpec((BLK, D), lambda i: (i, 0)),
        out_shape=jax.ShapeDtypeStruct((N, D), jnp.float32),
    )(acc, cnt3, q)


def kernel(x, edge_index, W1l, b1, W1r, W2l, b2, W2r):
    ei = edge_index.astype(jnp.int32)
    pad = EPAD - E
    src = jnp.concatenate([ei[0], jnp.zeros((pad,), jnp.int32)])
    dst = jnp.concatenate([ei[1], jnp.full((pad,), N, jnp.int32)])
    srcw = src.reshape(NW, NCH, CH)
    dstw = dst.reshape(NW, NCH, CH)
    b1r = b1.reshape(1, D)
    b2r = b2.reshape(1, D)

    p1, r1 = _pre(x, W1l, W1r, b1r)
    acc1, cnt = _sc_scatter_cnt(p1, srcw, dstw)
    cnt3 = cnt.reshape(NC, NACC, 1)
    p2, r2 = _mid(acc1, cnt3, r1, W2l, W2r, b2r)
    acc2 = _sc_scatter_nocnt(p2, srcw, dstw)
    return _post(acc2, cnt3, r2)
